# baseline (speedup 1.0000x reference)
import jax
import jax.numpy as jnp
from jax.experimental import pallas as pl
from jax.experimental.pallas import tpu as pltpu

D = 2048
H = 4096
E = 8
K = 2
N = 2048
ENTROPY_WEIGHT = 0.1
NT = 8
TBLK = N // NT
HC = H // NT
_BIG = 1 << 20
_HIGH = jax.lax.Precision.HIGHEST


def _router_body(x8_ref, gw_ref, gb_ref, coef_ref, esel_ref):
    logits = jax.lax.dot_general(
        x8_ref[...], gw_ref[...], (((1,), (1,)), ((), ())),
        preferred_element_type=jnp.float32, precision=_HIGH,
    ) + gb_ref[...]
    m = jnp.max(logits, axis=-1, keepdims=True)
    ex = jnp.exp(logits - m)
    p8 = ex / jnp.sum(ex, axis=-1, keepdims=True)
    col = jax.lax.broadcasted_iota(jnp.int32, (8, E), 1)
    row = jax.lax.broadcasted_iota(jnp.int32, (8, 1), 0)
    v1 = jnp.max(p8, axis=-1, keepdims=True)
    i1 = jnp.min(jnp.where(p8 == v1, col, E), axis=-1, keepdims=True)
    p8b = jnp.where(col == i1, -jnp.inf, p8)
    v2 = jnp.max(p8b, axis=-1, keepdims=True)
    i2 = jnp.min(jnp.where(p8b == v2, col, E), axis=-1, keepdims=True)
    for k, (vk, ik) in enumerate(((v1, i1), (v2, i2))):
        mask = ik == row
        coef_ref[k, :, :] = jnp.where(mask, vk, 0.0)
        first = jnp.min(jnp.where(mask, row, _BIG))
        rowsel = jnp.where(first == _BIG, 0, first)
        esel_ref[k] = jnp.sum(jnp.where(row == rowsel, ik, 0))


def _fused_body(esel_ref, x8_ref, coef_ref, w_ref, b_ref, xe_ref, gw_ref,
                gb_ref, out_ref, ent_ref, acc_ref):
    t = pl.program_id(0)
    k = pl.program_id(1)

    y = jax.lax.dot_general(
        x8_ref[...], w_ref[0], (((1,), (1,)), ((), ())),
        preferred_element_type=jnp.float32, precision=_HIGH,
    )
    y = (y + b_ref[0]) * coef_ref[0]

    @pl.when(k == 0)
    def _():
        acc_ref[:, pl.ds(t * HC, HC)] = y

    @pl.when(k == 1)
    def _():
        acc_ref[:, pl.ds(t * HC, HC)] += y

    @pl.when(k == 0)
    def _():
        logits = jax.lax.dot_general(
            xe_ref[...], gw_ref[...], (((1,), (1,)), ((), ())),
            preferred_element_type=jnp.float32, precision=_HIGH,
        ) + gb_ref[...]
        m = jnp.max(logits, axis=-1, keepdims=True)
        exl = jnp.exp(logits - m)
        p = exl / jnp.sum(exl, axis=-1, keepdims=True)
        ent = -jnp.sum(p * jnp.log(p + 1e-10))

        @pl.when(t == 0)
        def _():
            ent_ref[0, 0] = ent

        @pl.when(t != 0)
        def _():
            ent_ref[0, 0] += ent

    @pl.when(k == 1)
    def _():
        out_ref[...] = jnp.zeros_like(out_ref)

        @pl.when(t == NT - 1)
        def _():
            out_ref[0:8, :] = acc_ref[...]


def kernel(x, gate_W, gate_b, expert_W, expert_b):
    x_flat = x.reshape(N, D)
    x8 = x_flat[0:8]
    gb = gate_b.reshape(1, E)
    coef, esel = pl.pallas_call(
        _router_body,
        in_specs=[
            pl.BlockSpec((8, D), lambda: (0, 0)),
            pl.BlockSpec((E, D), lambda: (0, 0)),
            pl.BlockSpec((1, E), lambda: (0, 0)),
        ],
        out_specs=[
            pl.BlockSpec((K, 8, 1), lambda: (0, 0, 0)),
            pl.BlockSpec(memory_space=pltpu.SMEM),
        ],
        out_shape=[
            jax.ShapeDtypeStruct((K, 8, 1), jnp.float32),
            jax.ShapeDtypeStruct((K,), jnp.int32),
        ],
    )(x8, gate_W, gb)

    out, ent = pl.pallas_call(
        _fused_body,
        grid_spec=pltpu.PrefetchScalarGridSpec(
            num_scalar_prefetch=1,
            grid=(NT, K),
            in_specs=[
                pl.BlockSpec((8, D), lambda t, k, s: (0, 0)),
                pl.BlockSpec((1, 8, 1), lambda t, k, s: (k, 0, 0)),
                pl.BlockSpec((1, HC, D), lambda t, k, s: (s[k], t, 0)),
                pl.BlockSpec((1, 1, HC), lambda t, k, s: (s[k], 0, t)),
                pl.BlockSpec((TBLK, D), lambda t, k, s: (NT - 1 - t, 0)),
                pl.BlockSpec((E, D), lambda t, k, s: (0, 0)),
                pl.BlockSpec((1, E), lambda t, k, s: (0, 0)),
            ],
            out_specs=[
                pl.BlockSpec((TBLK, H), lambda t, k, s: (NT - 1 - t, 0)),
                pl.BlockSpec(memory_space=pltpu.SMEM),
            ],
            scratch_shapes=[pltpu.VMEM((8, H), jnp.float32)],
        ),
        out_shape=[
            jax.ShapeDtypeStruct((N, H), jnp.float32),
            jax.ShapeDtypeStruct((1, 1), jnp.float32),
        ],
    )(esel, x8, coef, expert_W, expert_b.reshape(E, 1, H), x_flat,
      gate_W, gb)

    loss = ENTROPY_WEIGHT * ent[0, 0] / N
    return out.reshape(1, N, H), loss


# speedup vs baseline: 1.4086x; 1.4086x over previous
import jax
import jax.numpy as jnp
from jax.experimental import pallas as pl
from jax.experimental.pallas import tpu as pltpu

D = 2048
H = 4096
E = 8
K = 2
N = 2048
ENTROPY_WEIGHT = 0.1
NT = 8
TBLK = N // NT
HC = H // NT
_BIG = 1 << 20
_HIGH = jax.lax.Precision.HIGHEST


def _router_body(x8_ref, gw_ref, gb_ref, coef_ref, esel_ref):
    logits = jax.lax.dot_general(
        x8_ref[...], gw_ref[...], (((1,), (1,)), ((), ())),
        preferred_element_type=jnp.float32, precision=_HIGH,
    ) + gb_ref[...]
    m = jnp.max(logits, axis=-1, keepdims=True)
    ex = jnp.exp(logits - m)
    p8 = ex / jnp.sum(ex, axis=-1, keepdims=True)
    col = jax.lax.broadcasted_iota(jnp.int32, (8, E), 1)
    row = jax.lax.broadcasted_iota(jnp.int32, (8, 1), 0)
    v1 = jnp.max(p8, axis=-1, keepdims=True)
    i1 = jnp.min(jnp.where(p8 == v1, col, E), axis=-1, keepdims=True)
    p8b = jnp.where(col == i1, -jnp.inf, p8)
    v2 = jnp.max(p8b, axis=-1, keepdims=True)
    i2 = jnp.min(jnp.where(p8b == v2, col, E), axis=-1, keepdims=True)
    for k, (vk, ik) in enumerate(((v1, i1), (v2, i2))):
        mask = ik == row
        coef_ref[k, :, :] = jnp.where(mask, vk, 0.0)
        first = jnp.min(jnp.where(mask, row, _BIG))
        rowsel = jnp.where(first == _BIG, 0, first)
        esel_ref[k] = jnp.sum(jnp.where(row == rowsel, ik, 0))


def _fused_body(esel_ref, x8_ref, coef_ref, w_ref, b_ref, xe_ref, gw_ref,
                gb_ref, out_ref, ent_ref, acc_ref):
    t = pl.program_id(0)
    k = pl.program_id(1)

    y = jax.lax.dot_general(
        x8_ref[...], w_ref[0], (((1,), (1,)), ((), ())),
        preferred_element_type=jnp.float32,
    )
    y = (y + b_ref[0]) * coef_ref[0]

    @pl.when(k == 0)
    def _():
        acc_ref[:, pl.ds(t * HC, HC)] = y

    @pl.when(k == 1)
    def _():
        acc_ref[:, pl.ds(t * HC, HC)] += y

    @pl.when(k == 0)
    def _():
        logits = jax.lax.dot_general(
            xe_ref[...], gw_ref[...], (((1,), (1,)), ((), ())),
            preferred_element_type=jnp.float32, precision=_HIGH,
        ) + gb_ref[...]
        m = jnp.max(logits, axis=-1, keepdims=True)
        exl = jnp.exp(logits - m)
        p = exl / jnp.sum(exl, axis=-1, keepdims=True)
        ent = -jnp.sum(p * jnp.log(p + 1e-10))

        @pl.when(t == 0)
        def _():
            ent_ref[0, 0] = ent

        @pl.when(t != 0)
        def _():
            ent_ref[0, 0] += ent

    @pl.when(k == 1)
    def _():
        out_ref[...] = jnp.zeros_like(out_ref)

        @pl.when(t == NT - 1)
        def _():
            out_ref[0:8, :] = acc_ref[...]


def kernel(x, gate_W, gate_b, expert_W, expert_b):
    x_flat = x.reshape(N, D)
    x8 = x_flat[0:8]
    gb = gate_b.reshape(1, E)
    coef, esel = pl.pallas_call(
        _router_body,
        in_specs=[
            pl.BlockSpec((8, D), lambda: (0, 0)),
            pl.BlockSpec((E, D), lambda: (0, 0)),
            pl.BlockSpec((1, E), lambda: (0, 0)),
        ],
        out_specs=[
            pl.BlockSpec((K, 8, 1), lambda: (0, 0, 0)),
            pl.BlockSpec(memory_space=pltpu.SMEM),
        ],
        out_shape=[
            jax.ShapeDtypeStruct((K, 8, 1), jnp.float32),
            jax.ShapeDtypeStruct((K,), jnp.int32),
        ],
    )(x8, gate_W, gb)

    out, ent = pl.pallas_call(
        _fused_body,
        grid_spec=pltpu.PrefetchScalarGridSpec(
            num_scalar_prefetch=1,
            grid=(NT, K),
            in_specs=[
                pl.BlockSpec((8, D), lambda t, k, s: (0, 0)),
                pl.BlockSpec((1, 8, 1), lambda t, k, s: (k, 0, 0)),
                pl.BlockSpec((1, HC, D), lambda t, k, s: (s[k], t, 0)),
                pl.BlockSpec((1, 1, HC), lambda t, k, s: (s[k], 0, t)),
                pl.BlockSpec((TBLK, D), lambda t, k, s: (NT - 1 - t, 0)),
                pl.BlockSpec((E, D), lambda t, k, s: (0, 0)),
                pl.BlockSpec((1, E), lambda t, k, s: (0, 0)),
            ],
            out_specs=[
                pl.BlockSpec((TBLK, H), lambda t, k, s: (NT - 1 - t, 0)),
                pl.BlockSpec(memory_space=pltpu.SMEM),
            ],
            scratch_shapes=[pltpu.VMEM((8, H), jnp.float32)],
        ),
        out_shape=[
            jax.ShapeDtypeStruct((N, H), jnp.float32),
            jax.ShapeDtypeStruct((1, 1), jnp.float32),
        ],
    )(esel, x8, coef, expert_W, expert_b.reshape(E, 1, H), x_flat,
      gate_W, gb)

    loss = ENTROPY_WEIGHT * ent[0, 0] / N
    return out.reshape(1, N, H), loss


# speedup vs baseline: 1.6515x; 1.1725x over previous
import jax
import jax.numpy as jnp
from jax.experimental import pallas as pl
from jax.experimental.pallas import tpu as pltpu

D = 2048
H = 4096
E = 8
K = 2
N = 2048
ENTROPY_WEIGHT = 0.1
NT = 8
TBLK = N // NT
HC = H // NT
_BIG = 1 << 20
_HIGH = jax.lax.Precision.HIGHEST


def _router_body(x8_ref, gw_ref, gb_ref, coef_ref, esel_ref):
    logits = jax.lax.dot_general(
        x8_ref[...], gw_ref[...], (((1,), (1,)), ((), ())),
        preferred_element_type=jnp.float32, precision=_HIGH,
    ) + gb_ref[...]
    m = jnp.max(logits, axis=-1, keepdims=True)
    ex = jnp.exp(logits - m)
    p8 = ex / jnp.sum(ex, axis=-1, keepdims=True)
    col = jax.lax.broadcasted_iota(jnp.int32, (8, E), 1)
    row = jax.lax.broadcasted_iota(jnp.int32, (8, 1), 0)
    v1 = jnp.max(p8, axis=-1, keepdims=True)
    i1 = jnp.min(jnp.where(p8 == v1, col, E), axis=-1, keepdims=True)
    p8b = jnp.where(col == i1, -jnp.inf, p8)
    v2 = jnp.max(p8b, axis=-1, keepdims=True)
    i2 = jnp.min(jnp.where(p8b == v2, col, E), axis=-1, keepdims=True)
    for k, (vk, ik) in enumerate(((v1, i1), (v2, i2))):
        mask = ik == row
        coef_ref[k, :, :] = jnp.where(mask, vk, 0.0)
        first = jnp.min(jnp.where(mask, row, _BIG))
        rowsel = jnp.where(first == _BIG, 0, first)
        esel_ref[k] = jnp.sum(jnp.where(row == rowsel, ik, 0))


def _fused_body(esel_ref, x8_ref, coef_ref, w0_ref, w1_ref, b0_ref, b1_ref,
                xe_ref, gw_ref, gb_ref, out_ref, ent_ref, acc_ref):
    t = pl.program_id(0)

    y0 = jax.lax.dot_general(
        x8_ref[...], w0_ref[0], (((1,), (1,)), ((), ())),
        preferred_element_type=jnp.float32,
    )
    y1 = jax.lax.dot_general(
        x8_ref[...], w1_ref[0], (((1,), (1,)), ((), ())),
        preferred_element_type=jnp.float32,
    )
    y = ((y0 + b0_ref[0]) * coef_ref[0, :, :]
         + (y1 + b1_ref[0]) * coef_ref[1, :, :])
    acc_ref[:, pl.ds(t * HC, HC)] = y

    logits = jax.lax.dot_general(
        xe_ref[...], gw_ref[...], (((1,), (1,)), ((), ())),
        preferred_element_type=jnp.float32, precision=_HIGH,
    ) + gb_ref[...]
    m = jnp.max(logits, axis=-1, keepdims=True)
    exl = jnp.exp(logits - m)
    p = exl / jnp.sum(exl, axis=-1, keepdims=True)
    ent = -jnp.sum(p * jnp.log(p + 1e-10))

    @pl.when(t == 0)
    def _():
        ent_ref[0, 0] = ent

    @pl.when(t != 0)
    def _():
        ent_ref[0, 0] += ent

    out_ref[...] = jnp.zeros_like(out_ref)

    @pl.when(t == NT - 1)
    def _():
        out_ref[0:8, :] = acc_ref[...]


def kernel(x, gate_W, gate_b, expert_W, expert_b):
    x_flat = x.reshape(N, D)
    x8 = x_flat[0:8]
    gb = gate_b.reshape(1, E)
    coef, esel = pl.pallas_call(
        _router_body,
        in_specs=[
            pl.BlockSpec((8, D), lambda: (0, 0)),
            pl.BlockSpec((E, D), lambda: (0, 0)),
            pl.BlockSpec((1, E), lambda: (0, 0)),
        ],
        out_specs=[
            pl.BlockSpec((K, 8, 1), lambda: (0, 0, 0)),
            pl.BlockSpec(memory_space=pltpu.SMEM),
        ],
        out_shape=[
            jax.ShapeDtypeStruct((K, 8, 1), jnp.float32),
            jax.ShapeDtypeStruct((K,), jnp.int32),
        ],
    )(x8, gate_W, gb)

    out, ent = pl.pallas_call(
        _fused_body,
        grid_spec=pltpu.PrefetchScalarGridSpec(
            num_scalar_prefetch=1,
            grid=(NT,),
            in_specs=[
                pl.BlockSpec((8, D), lambda t, s: (0, 0)),
                pl.BlockSpec((K, 8, 1), lambda t, s: (0, 0, 0)),
                pl.BlockSpec((1, HC, D), lambda t, s: (s[0], t, 0)),
                pl.BlockSpec((1, HC, D), lambda t, s: (s[1], t, 0)),
                pl.BlockSpec((1, 1, HC), lambda t, s: (s[0], 0, t)),
                pl.BlockSpec((1, 1, HC), lambda t, s: (s[1], 0, t)),
                pl.BlockSpec((TBLK, D), lambda t, s: (NT - 1 - t, 0)),
                pl.BlockSpec((E, D), lambda t, s: (0, 0)),
                pl.BlockSpec((1, E), lambda t, s: (0, 0)),
            ],
            out_specs=[
                pl.BlockSpec((TBLK, H), lambda t, s: (NT - 1 - t, 0)),
                pl.BlockSpec(memory_space=pltpu.SMEM),
            ],
            scratch_shapes=[pltpu.VMEM((8, H), jnp.float32)],
        ),
        out_shape=[
            jax.ShapeDtypeStruct((N, H), jnp.float32),
            jax.ShapeDtypeStruct((1, 1), jnp.float32),
        ],
    )(esel, x8, coef, expert_W, expert_W,
      expert_b.reshape(E, 1, H), expert_b.reshape(E, 1, H), x_flat,
      gate_W, gb)

    loss = ENTROPY_WEIGHT * ent[0, 0] / N
    return out.reshape(1, N, H), loss


# speedup vs baseline: 1.7922x; 1.0852x over previous
import jax
import jax.numpy as jnp
from jax.experimental import pallas as pl
from jax.experimental.pallas import tpu as pltpu

D = 2048
H = 4096
E = 8
K = 2
N = 2048
ENTROPY_WEIGHT = 0.1
NT = 8
TBLK = N // NT
HC = H // NT
_BIG = 1 << 20
_HIGH = jax.lax.Precision.HIGHEST


def _router_body(x8_ref, gw_ref, gb_ref, coef_ref, esel_ref):
    logits = jax.lax.dot_general(
        x8_ref[...], gw_ref[...], (((1,), (1,)), ((), ())),
        preferred_element_type=jnp.float32, precision=_HIGH,
    ) + gb_ref[...]
    m = jnp.max(logits, axis=-1, keepdims=True)
    ex = jnp.exp(logits - m)
    p8 = ex / jnp.sum(ex, axis=-1, keepdims=True)
    col = jax.lax.broadcasted_iota(jnp.int32, (8, E), 1)
    row = jax.lax.broadcasted_iota(jnp.int32, (8, 1), 0)
    v1 = jnp.max(p8, axis=-1, keepdims=True)
    i1 = jnp.min(jnp.where(p8 == v1, col, E), axis=-1, keepdims=True)
    p8b = jnp.where(col == i1, -jnp.inf, p8)
    v2 = jnp.max(p8b, axis=-1, keepdims=True)
    i2 = jnp.min(jnp.where(p8b == v2, col, E), axis=-1, keepdims=True)
    for k, (vk, ik) in enumerate(((v1, i1), (v2, i2))):
        mask = ik == row
        coef_ref[k, :, :] = jnp.where(mask, vk, 0.0)
        first = jnp.min(jnp.where(mask, row, _BIG))
        rowsel = jnp.where(first == _BIG, 0, first)
        esel_ref[k] = jnp.sum(jnp.where(row == rowsel, ik, 0))


def _fused_body(esel_ref, x8_ref, coef_ref, w0_ref, w1_ref, b0_ref, b1_ref,
                xe_ref, gw_ref, gb_ref, out_ref, ent_ref, acc_ref):
    t = pl.program_id(0)

    y0 = jax.lax.dot_general(
        x8_ref[...], w0_ref[0], (((1,), (1,)), ((), ())),
        preferred_element_type=jnp.float32,
    )
    y1 = jax.lax.dot_general(
        x8_ref[...], w1_ref[0], (((1,), (1,)), ((), ())),
        preferred_element_type=jnp.float32,
    )
    y = ((y0 + b0_ref[0]) * coef_ref[0, :, :]
         + (y1 + b1_ref[0]) * coef_ref[1, :, :])
    acc_ref[:, pl.ds(t * HC, HC)] = y

    logits = jax.lax.dot_general(
        xe_ref[...], gw_ref[...], (((1,), (1,)), ((), ())),
        preferred_element_type=jnp.float32,
    ) + gb_ref[...]
    m = jnp.max(logits, axis=-1, keepdims=True)
    exl = jnp.exp(logits - m)
    p = exl / jnp.sum(exl, axis=-1, keepdims=True)
    ent = -jnp.sum(p * jnp.log(p + 1e-10))

    @pl.when(t == 0)
    def _():
        ent_ref[0, 0] = ent

    @pl.when(t != 0)
    def _():
        ent_ref[0, 0] += ent

    out_ref[...] = jnp.zeros_like(out_ref)

    @pl.when(t == NT - 1)
    def _():
        out_ref[0:8, :] = acc_ref[...]


def kernel(x, gate_W, gate_b, expert_W, expert_b):
    x_flat = x.reshape(N, D)
    x8 = x_flat[0:8]
    gb = gate_b.reshape(1, E)
    coef, esel = pl.pallas_call(
        _router_body,
        in_specs=[
            pl.BlockSpec((8, D), lambda: (0, 0)),
            pl.BlockSpec((E, D), lambda: (0, 0)),
            pl.BlockSpec((1, E), lambda: (0, 0)),
        ],
        out_specs=[
            pl.BlockSpec((K, 8, 1), lambda: (0, 0, 0)),
            pl.BlockSpec(memory_space=pltpu.SMEM),
        ],
        out_shape=[
            jax.ShapeDtypeStruct((K, 8, 1), jnp.float32),
            jax.ShapeDtypeStruct((K,), jnp.int32),
        ],
    )(x8, gate_W, gb)

    out, ent = pl.pallas_call(
        _fused_body,
        grid_spec=pltpu.PrefetchScalarGridSpec(
            num_scalar_prefetch=1,
            grid=(NT,),
            in_specs=[
                pl.BlockSpec((8, D), lambda t, s: (0, 0)),
                pl.BlockSpec((K, 8, 1), lambda t, s: (0, 0, 0)),
                pl.BlockSpec((1, HC, D), lambda t, s: (s[0], t, 0)),
                pl.BlockSpec((1, HC, D), lambda t, s: (s[1], t, 0)),
                pl.BlockSpec((1, 1, HC), lambda t, s: (s[0], 0, t)),
                pl.BlockSpec((1, 1, HC), lambda t, s: (s[1], 0, t)),
                pl.BlockSpec((TBLK, D), lambda t, s: (NT - 1 - t, 0)),
                pl.BlockSpec((E, D), lambda t, s: (0, 0)),
                pl.BlockSpec((1, E), lambda t, s: (0, 0)),
            ],
            out_specs=[
                pl.BlockSpec((TBLK, H), lambda t, s: (NT - 1 - t, 0)),
                pl.BlockSpec(memory_space=pltpu.SMEM),
            ],
            scratch_shapes=[pltpu.VMEM((8, H), jnp.float32)],
        ),
        out_shape=[
            jax.ShapeDtypeStruct((N, H), jnp.float32),
            jax.ShapeDtypeStruct((1, 1), jnp.float32),
        ],
    )(esel, x8, coef, expert_W, expert_W,
      expert_b.reshape(E, 1, H), expert_b.reshape(E, 1, H), x_flat,
      gate_W, gb)

    loss = ENTROPY_WEIGHT * ent[0, 0] / N
    return out.reshape(1, N, H), loss


# speedup vs baseline: 1.7972x; 1.0028x over previous
import jax
import jax.numpy as jnp
from jax.experimental import pallas as pl
from jax.experimental.pallas import tpu as pltpu

D = 2048
H = 4096
E = 8
K = 2
N = 2048
ENTROPY_WEIGHT = 0.1
NT = 8
TBLK = N // NT
HC = H // NT
_BIG = 1 << 20
_HIGH = jax.lax.Precision.HIGHEST


def _router_body(x8_ref, gw_ref, gb_ref, coef_ref, esel_ref):
    logits = jax.lax.dot_general(
        x8_ref[...], gw_ref[...], (((1,), (1,)), ((), ())),
        preferred_element_type=jnp.float32, precision=_HIGH,
    ) + gb_ref[...]
    m = jnp.max(logits, axis=-1, keepdims=True)
    ex = jnp.exp(logits - m)
    p8 = ex / jnp.sum(ex, axis=-1, keepdims=True)
    col = jax.lax.broadcasted_iota(jnp.int32, (8, E), 1)
    row = jax.lax.broadcasted_iota(jnp.int32, (8, 1), 0)
    v1 = jnp.max(p8, axis=-1, keepdims=True)
    i1 = jnp.min(jnp.where(p8 == v1, col, E), axis=-1, keepdims=True)
    p8b = jnp.where(col == i1, -jnp.inf, p8)
    v2 = jnp.max(p8b, axis=-1, keepdims=True)
    i2 = jnp.min(jnp.where(p8b == v2, col, E), axis=-1, keepdims=True)
    for k, (vk, ik) in enumerate(((v1, i1), (v2, i2))):
        mask = ik == row
        coef_ref[k, :, :] = jnp.where(mask, vk, 0.0)
        first = jnp.min(jnp.where(mask, row, _BIG))
        rowsel = jnp.where(first == _BIG, 0, first)
        esel_ref[k] = jnp.sum(jnp.where(row == rowsel, ik, 0))


def _fused_body(esel_ref, x8_ref, coef_ref, w0a_ref, w0b_ref, w1a_ref,
                w1b_ref, b0_ref, b1_ref, xe_ref, gw_ref, gb_ref, out_ref,
                ent_ref, acc_ref):
    t = pl.program_id(0)

    for half, (wa, wb) in enumerate(((w0a_ref, w1a_ref), (w0b_ref, w1b_ref))):
        y0 = jax.lax.dot_general(
            x8_ref[...], wa[0], (((1,), (1,)), ((), ())),
            preferred_element_type=jnp.float32,
        )
        y1 = jax.lax.dot_general(
            x8_ref[...], wb[0], (((1,), (1,)), ((), ())),
            preferred_element_type=jnp.float32,
        )
        bsl = pl.ds(half * (HC // 2), HC // 2)
        y = ((y0 + b0_ref[0, :, bsl]) * coef_ref[0, :, :]
             + (y1 + b1_ref[0, :, bsl]) * coef_ref[1, :, :])
        acc_ref[:, pl.ds(t * HC + half * (HC // 2), HC // 2)] = y

    logits = jax.lax.dot_general(
        xe_ref[...], gw_ref[...], (((1,), (1,)), ((), ())),
        preferred_element_type=jnp.float32,
    ) + gb_ref[...]
    m = jnp.max(logits, axis=-1, keepdims=True)
    exl = jnp.exp(logits - m)
    p = exl / jnp.sum(exl, axis=-1, keepdims=True)
    ent = -jnp.sum(p * jnp.log(p + 1e-10))

    @pl.when(t == 0)
    def _():
        ent_ref[0, 0] = ent

    @pl.when(t != 0)
    def _():
        ent_ref[0, 0] += ent

    out_ref[...] = jnp.zeros_like(out_ref)

    @pl.when(t == NT - 1)
    def _():
        out_ref[0:8, :] = acc_ref[...]


def kernel(x, gate_W, gate_b, expert_W, expert_b):
    x_flat = x.reshape(N, D)
    x8 = x_flat[0:8]
    gb = gate_b.reshape(1, E)
    coef, esel = pl.pallas_call(
        _router_body,
        in_specs=[
            pl.BlockSpec((8, D), lambda: (0, 0)),
            pl.BlockSpec((E, D), lambda: (0, 0)),
            pl.BlockSpec((1, E), lambda: (0, 0)),
        ],
        out_specs=[
            pl.BlockSpec((K, 8, 1), lambda: (0, 0, 0)),
            pl.BlockSpec(memory_space=pltpu.SMEM),
        ],
        out_shape=[
            jax.ShapeDtypeStruct((K, 8, 1), jnp.float32),
            jax.ShapeDtypeStruct((K,), jnp.int32),
        ],
    )(x8, gate_W, gb)

    out, ent = pl.pallas_call(
        _fused_body,
        grid_spec=pltpu.PrefetchScalarGridSpec(
            num_scalar_prefetch=1,
            grid=(NT,),
            in_specs=[
                pl.BlockSpec((8, D), lambda t, s: (0, 0)),
                pl.BlockSpec((K, 8, 1), lambda t, s: (0, 0, 0)),
                pl.BlockSpec((1, HC // 2, D), lambda t, s: (s[0], 2 * t, 0)),
                pl.BlockSpec((1, HC // 2, D), lambda t, s: (s[0], 2 * t + 1, 0)),
                pl.BlockSpec((1, HC // 2, D), lambda t, s: (s[1], 2 * t, 0)),
                pl.BlockSpec((1, HC // 2, D), lambda t, s: (s[1], 2 * t + 1, 0)),
                pl.BlockSpec((1, 1, HC), lambda t, s: (s[0], 0, t)),
                pl.BlockSpec((1, 1, HC), lambda t, s: (s[1], 0, t)),
                pl.BlockSpec((TBLK, D), lambda t, s: (NT - 1 - t, 0)),
                pl.BlockSpec((E, D), lambda t, s: (0, 0)),
                pl.BlockSpec((1, E), lambda t, s: (0, 0)),
            ],
            out_specs=[
                pl.BlockSpec((TBLK, H), lambda t, s: (NT - 1 - t, 0)),
                pl.BlockSpec(memory_space=pltpu.SMEM),
            ],
            scratch_shapes=[pltpu.VMEM((8, H), jnp.float32)],
        ),
        out_shape=[
            jax.ShapeDtypeStruct((N, H), jnp.float32),
            jax.ShapeDtypeStruct((1, 1), jnp.float32),
        ],
    )(esel, x8, coef, expert_W, expert_W, expert_W, expert_W,
      expert_b.reshape(E, 1, H), expert_b.reshape(E, 1, H), x_flat,
      gate_W, gb)

    loss = ENTROPY_WEIGHT * ent[0, 0] / N
    return out.reshape(1, N, H), loss
